# baseline (speedup 1.0000x reference)
import jax
import jax.numpy as jnp
from jax import lax
from jax.experimental import pallas as pl
from jax.experimental.pallas import tpu as pltpu
from jax.experimental.pallas import tpu_sc as plsc

NC, NS, L = 2, 16, 16
NW = NC * NS
B, S, H = 32, 1024, 1024
SW = S // NW
R = 16
CPB = SW // R
NK = B * CPB


def _emb_body(x_hbm, pix_hbm, pos_hbm, out_hbm,
              idx_v, pos_v, g0, g1, g2, g3,
              gsem0, gsem1, gsem2, gsem3,
              stsem0, stsem1, stsem2, stsem3):
    wid = lax.axis_index("s") * NC + lax.axis_index("c")
    col0 = pl.multiple_of(wid * SW, SW)
    xblk = pl.multiple_of((wid // 4) * 128, 128)
    coff = (wid % 4) * SW
    G = (g0, g1, g2, g3)
    GSEM = (gsem0, gsem1, gsem2, gsem3)
    STSEM = (stsem0, stsem1, stsem2, stsem3)

    pltpu.sync_copy(x_hbm.at[:, pl.ds(xblk, 128)], idx_v)
    pltpu.sync_copy(pos_hbm.at[pl.ds(col0, SW), :], pos_v)

    def chunk_bs(kk, slot):
        b = kk // CPB + slot // CPB
        srow = (slot % CPB) * R
        return b, srow

    def start_gather(b, srow, slot):
        pltpu.async_copy(
            pix_hbm.at[idx_v.at[b, pl.ds(coff + srow, R)]], G[slot],
            GSEM[slot],
        )

    def wait_gather(b, srow, slot):
        pltpu.make_async_copy(
            pix_hbm.at[idx_v.at[b, pl.ds(coff + srow, R)]], G[slot],
            GSEM[slot],
        ).wait()

    def wait_store(b, srow, slot):
        pltpu.make_async_copy(
            G[slot], out_hbm.at[b, pl.ds(col0 + srow, R), :], STSEM[slot]
        ).wait()

    start_gather(0, 0, 0)
    start_gather(0, R, 1)

    def step(i, carry):
        kk = i * 4
        for slot in range(4):
            b, srow = chunk_bs(kk, slot)
            gbuf = G[slot]
            wait_gather(b, srow, slot)
            def add_row(r, c2, _gbuf=gbuf, _srow=srow):
                for u in range(H // L):
                    cs = pl.ds(u * L, L)
                    plsc.addupdate(_gbuf.at[r, cs], pos_v[_srow + r, cs])
                return c2
            lax.fori_loop(0, R, add_row, 0, unroll=False)
            nslot = (slot + 2) % 4
            nb = kk // 2 + (slot + 2) // 2
            pb = kk // 2 + (slot - 2) // 2

            @pl.when(nb < B)
            def _():
                @pl.when((kk > 0) | (slot >= 2))
                def _():
                    wait_store(pb, srow, nslot)
                start_gather(nb, srow, nslot)
            pltpu.async_copy(
                gbuf, out_hbm.at[b, pl.ds(col0 + srow, R), :], STSEM[slot]
            )
        return carry

    lax.fori_loop(0, NK // 4, step, 0, unroll=False)

    for slot in range(4):
        b, srow = chunk_bs(NK - 4, slot)
        wait_store(b, srow, slot)


@jax.jit
def _emb(x, pix_table, pos_table):
    run = pl.kernel(
        _emb_body,
        out_type=jax.ShapeDtypeStruct((B, S, H), jnp.float32),
        mesh=plsc.VectorSubcoreMesh(core_axis_name="c", subcore_axis_name="s"),
        scratch_types=[
            pltpu.VMEM((B, 128), jnp.int32),
            pltpu.VMEM((SW, H), jnp.float32),
            pltpu.VMEM((R, H), jnp.float32),
            pltpu.VMEM((R, H), jnp.float32),
            pltpu.VMEM((R, H), jnp.float32),
            pltpu.VMEM((R, H), jnp.float32),
            pltpu.SemaphoreType.DMA,
            pltpu.SemaphoreType.DMA,
            pltpu.SemaphoreType.DMA,
            pltpu.SemaphoreType.DMA,
            pltpu.SemaphoreType.DMA,
            pltpu.SemaphoreType.DMA,
            pltpu.SemaphoreType.DMA,
            pltpu.SemaphoreType.DMA,
        ],
    )
    return run(x, pix_table, pos_table)


def kernel(x, pix_table, pos_table):
    return _emb(x, pix_table, pos_table)
